# baseline (speedup 1.0000x reference)
import functools

import jax
import jax.numpy as jnp
from jax import lax
from jax.experimental import pallas as pl
from jax.experimental.pallas import tpu as pltpu
from jax.experimental.pallas import tpu_sc as plsc

_N = 10000
_H = 128
_SUPPORT = 2
_NUM_BASES = 2
_NUM_IDX = 1024

_NC = 2
_NS = 16
_NW = _NC * _NS
_BPW = _NUM_IDX // _NW
_CW = 768
_NCH = 13
_TW = 128
_AGW = _NCH * _CW + _TW
_NBUF = 4


def _sc_gather_body(a_hbm, at_hbm, idx_hbm, out_hbm, idxv,
                    b0, b1, b2, b3, buft,
                    g0, g1, g2, g3, s0, s1, s2, s3, semt):
    cid = lax.axis_index("c")
    sid = lax.axis_index("s")
    wid = sid * _NC + cid
    base = pl.multiple_of(wid * _BPW, _BPW)
    pltpu.sync_copy(idx_hbm.at[pl.ds(base, _BPW)], idxv)

    bufs = (b0, b1, b2, b3)
    gsems = (g0, g1, g2, g3)
    ssems = (s0, s1, s2, s3)
    jobs = [(j % 2, (j // 2) * _CW) for j in range(2 * _NCH)]

    def gather(j):
        s, off = jobs[j]
        return pltpu.async_copy(a_hbm.at[s].at[idxv, pl.ds(off, _CW)],
                                bufs[j % _NBUF], gsems[j % _NBUF])

    def scatter(j):
        s, off = jobs[j]
        return pltpu.async_copy(
            bufs[j % _NBUF],
            out_hbm.at[s, pl.ds(base, _BPW), pl.ds(off, _CW)],
            ssems[j % _NBUF])

    n = len(jobs)
    pend_g = {0: gather(0), 1: gather(1)}
    pend_s = {}
    for j in range(n):
        if j - 2 in pend_s:
            pend_s.pop(j - 2).wait()
        pend_g.pop(j).wait()
        pend_s[j] = scatter(j)
        if j + 2 < n:
            pend_g[j + 2] = gather(j + 2)
    for j, d in pend_s.items():
        d.wait()
    for s in range(_SUPPORT):
        pltpu.async_copy(at_hbm.at[s].at[idxv], buft, semt).wait()
        pltpu.sync_copy(buft, out_hbm.at[s, pl.ds(base, _BPW),
                                         pl.ds(_NCH * _CW, _TW)])


@functools.cache
def _sc_gather_kernel():
    return functools.partial(
        pl.kernel,
        mesh=plsc.VectorSubcoreMesh(core_axis_name="c", subcore_axis_name="s"),
        out_type=jax.ShapeDtypeStruct((_SUPPORT, _NUM_IDX, _AGW), jnp.float32),
        scratch_types=[
            pltpu.VMEM((_BPW,), jnp.int32),
            pltpu.VMEM((_BPW, _CW), jnp.float32),
            pltpu.VMEM((_BPW, _CW), jnp.float32),
            pltpu.VMEM((_BPW, _CW), jnp.float32),
            pltpu.VMEM((_BPW, _CW), jnp.float32),
            pltpu.VMEM((_BPW, _TW), jnp.float32),
            pltpu.SemaphoreType.DMA,
            pltpu.SemaphoreType.DMA,
            pltpu.SemaphoreType.DMA,
            pltpu.SemaphoreType.DMA,
            pltpu.SemaphoreType.DMA,
            pltpu.SemaphoreType.DMA,
            pltpu.SemaphoreType.DMA,
            pltpu.SemaphoreType.DMA,
            pltpu.SemaphoreType.DMA,
        ],
    )(_sc_gather_body)


def _sc_gather(a3, at, idxs):
    return _sc_gather_kernel()(a3, at, idxs)


_BM1 = 200


_BF = jnp.bfloat16


def _bdot(a, b):
    return jnp.dot(a.astype(_BF), b.astype(_BF),
                   preferred_element_type=jnp.float32)


def _comb(c0, v0, c1, v1):
    f32 = jnp.float32
    return (c0.astype(_BF).astype(f32) * v0.astype(_BF).astype(f32)
            + c1.astype(_BF).astype(f32) * v1.astype(_BF).astype(f32))


def _pass1_body(a0_ref, a1_ref, v1_ref, w1c_ref, b1_ref, gc1_ref, m_ref):
    i = pl.program_id(0)

    @pl.when(i == 0)
    def _init():
        for s in range(_SUPPORT):
            m_ref[s] = _comb(w1c_ref[s, 0], v1_ref[0],
                             w1c_ref[s, 1], v1_ref[1]).astype(_BF)

    acc = _bdot(a0_ref[0], m_ref[0])
    acc += _bdot(a1_ref[0], m_ref[1])
    gc1_ref[...] = jnp.maximum(acc + b1_ref[...], 0.0).astype(_BF)


def _pass1(A, V1, W1_comp, b1):
    grid = (_N // _BM1,)
    return pl.pallas_call(
        _pass1_body,
        grid=grid,
        in_specs=[
            pl.BlockSpec((1, _BM1, _N), lambda i: (0, i, 0)),
            pl.BlockSpec((1, _BM1, _N), lambda i: (1, i, 0)),
            pl.BlockSpec((_NUM_BASES, _N, _H), lambda i: (0, 0, 0)),
            pl.BlockSpec(memory_space=pltpu.SMEM),
            pl.BlockSpec((1, _H), lambda i: (0, 0)),
        ],
        out_specs=pl.BlockSpec((_BM1, _H), lambda i: (i, 0)),
        out_shape=jax.ShapeDtypeStruct((_N, _H), _BF),
        scratch_shapes=[
            pltpu.VMEM((_NUM_BASES, _N, _H), _BF),
        ],
        compiler_params=pltpu.CompilerParams(
            dimension_semantics=("arbitrary",)),
    )(A, A, V1, W1_comp, b1)


_BM2 = 256


def _pass2_body(ag0_ref, ag1_ref, gc1_ref, v2_ref, w2c_ref, b2_ref,
                wfc1_ref, bfc1_ref, wfc2_ref, bfc2_ref, out_ref):
    main = _NCH * _CW
    row = lax.broadcasted_iota(jnp.int32, (_TW, _H), 0)
    gt = jnp.where(row >= (_TW - (_N - main)), gc1_ref[_N - _TW:, :],
                   jnp.zeros((), _BF))
    e = jnp.zeros((_BM2, _H), jnp.float32)
    for s, ag_ref in ((0, ag0_ref), (1, ag1_ref)):
        ag = ag_ref[0]
        h = _bdot(ag[:, :main], gc1_ref[:main, :])
        h += _bdot(ag[:, main:], gt)
        w2e = _comb(w2c_ref[s, 0], v2_ref[0], w2c_ref[s, 1], v2_ref[1])
        e += _bdot(h, w2e)
    e = jnp.maximum(e + b2_ref[...], 0.0)
    f = jnp.maximum(_bdot(e, wfc1_ref[...]) + bfc1_ref[...], 0.0)
    logits = _bdot(f, wfc2_ref[...]) + bfc2_ref[...]
    mx = jnp.max(logits, axis=-1, keepdims=True)
    lse = mx + jnp.log(jnp.sum(jnp.exp(logits - mx), axis=-1, keepdims=True))
    out_ref[...] = logits - lse


def _pass2(Ag, gc1, V2, W2_comp, b2, Wfc1, bfc1, Wfc2, bfc2):
    grid = (_NUM_IDX // _BM2,)
    nclass = Wfc2.shape[1]
    return pl.pallas_call(
        _pass2_body,
        grid=grid,
        in_specs=[
            pl.BlockSpec((1, _BM2, _AGW), lambda i: (0, i, 0)),
            pl.BlockSpec((1, _BM2, _AGW), lambda i: (1, i, 0)),
            pl.BlockSpec((_N, _H), lambda i: (0, 0)),
            pl.BlockSpec((_NUM_BASES, _H, _H), lambda i: (0, 0, 0)),
            pl.BlockSpec(memory_space=pltpu.SMEM),
            pl.BlockSpec((1, _H), lambda i: (0, 0)),
            pl.BlockSpec((_H, _H), lambda i: (0, 0)),
            pl.BlockSpec((1, _H), lambda i: (0, 0)),
            pl.BlockSpec((_H, nclass), lambda i: (0, 0)),
            pl.BlockSpec((1, nclass), lambda i: (0, 0)),
        ],
        out_specs=pl.BlockSpec((_BM2, nclass), lambda i: (i, 0)),
        out_shape=jax.ShapeDtypeStruct((_NUM_IDX, nclass), jnp.float32),
        compiler_params=pltpu.CompilerParams(
            dimension_semantics=("arbitrary",)),
    )(Ag, Ag, gc1, V2, W2_comp, b2, Wfc1, bfc1, Wfc2, bfc2)


def kernel(vertex, A, idxs, W1, W1_comp, b1, W2, W2_comp, b2,
           Wfc1, bfc1, Wfc2, bfc2):
    del vertex
    idxs = idxs.astype(jnp.int32)
    nclass = Wfc2.shape[1]

    At = lax.slice(A, (0, 0, _N - _TW), (_SUPPORT, _N, _N))
    Ag = _sc_gather(A, At, idxs)
    V1 = W1.reshape(_NUM_BASES, _N, _H)
    V2 = W2.reshape(_NUM_BASES, _H, _H)
    gc1 = _pass1(A, V1, W1_comp, b1.reshape(1, _H))
    return _pass2(Ag, gc1, V2, W2_comp, b2.reshape(1, _H), Wfc1,
                  bfc1.reshape(1, _H), Wfc2, bfc2.reshape(1, nclass))


# speedup vs baseline: 1.0032x; 1.0032x over previous
import functools

import jax
import jax.numpy as jnp
from jax import lax
from jax.experimental import pallas as pl
from jax.experimental.pallas import tpu as pltpu
from jax.experimental.pallas import tpu_sc as plsc

_N = 10000
_H = 128
_SUPPORT = 2
_NUM_BASES = 2
_NUM_IDX = 1024

_NC = 2
_NS = 16
_NW = _NC * _NS
_BPW = _NUM_IDX // _NW
_CW = 768
_NCH = 13
_TW = 128
_AGW = _NCH * _CW + _TW
_NBUF = 4


def _sc_gather_body(a_hbm, at_hbm, idx_hbm, out_hbm, idxv,
                    b0, b1, b2, b3, buft,
                    g0, g1, g2, g3, s0, s1, s2, s3, semt):
    cid = lax.axis_index("c")
    sid = lax.axis_index("s")
    wid = sid * _NC + cid
    base = pl.multiple_of(wid * _BPW, _BPW)
    pltpu.sync_copy(idx_hbm.at[pl.ds(base, _BPW)], idxv)

    bufs = (b0, b1, b2, b3)
    gsems = (g0, g1, g2, g3)
    ssems = (s0, s1, s2, s3)
    jobs = [(j % 2, (j // 2) * _CW) for j in range(2 * _NCH)]

    def gather(j):
        s, off = jobs[j]
        return pltpu.async_copy(a_hbm.at[s].at[idxv, pl.ds(off, _CW)],
                                bufs[j % _NBUF], gsems[j % _NBUF])

    def scatter(j):
        s, off = jobs[j]
        return pltpu.async_copy(
            bufs[j % _NBUF],
            out_hbm.at[s, pl.ds(base, _BPW), pl.ds(off, _CW)],
            ssems[j % _NBUF])

    n = len(jobs)
    pend_g = {0: gather(0), 1: gather(1)}
    pend_s = {}
    for j in range(n):
        if j - 2 in pend_s:
            pend_s.pop(j - 2).wait()
        pend_g.pop(j).wait()
        pend_s[j] = scatter(j)
        if j + 2 < n:
            pend_g[j + 2] = gather(j + 2)
    for j, d in pend_s.items():
        d.wait()
    for s in range(_SUPPORT):
        pltpu.async_copy(at_hbm.at[s].at[idxv], buft, semt).wait()
        pltpu.sync_copy(buft, out_hbm.at[s, pl.ds(base, _BPW),
                                         pl.ds(_NCH * _CW, _TW)])


@functools.cache
def _sc_gather_kernel():
    return functools.partial(
        pl.kernel,
        mesh=plsc.VectorSubcoreMesh(core_axis_name="c", subcore_axis_name="s"),
        out_type=jax.ShapeDtypeStruct((_SUPPORT, _NUM_IDX, _AGW), jnp.float32),
        scratch_types=[
            pltpu.VMEM((_BPW,), jnp.int32),
            pltpu.VMEM((_BPW, _CW), jnp.float32),
            pltpu.VMEM((_BPW, _CW), jnp.float32),
            pltpu.VMEM((_BPW, _CW), jnp.float32),
            pltpu.VMEM((_BPW, _CW), jnp.float32),
            pltpu.VMEM((_BPW, _TW), jnp.float32),
            pltpu.SemaphoreType.DMA,
            pltpu.SemaphoreType.DMA,
            pltpu.SemaphoreType.DMA,
            pltpu.SemaphoreType.DMA,
            pltpu.SemaphoreType.DMA,
            pltpu.SemaphoreType.DMA,
            pltpu.SemaphoreType.DMA,
            pltpu.SemaphoreType.DMA,
            pltpu.SemaphoreType.DMA,
        ],
    )(_sc_gather_body)


def _sc_gather(a3, at, idxs):
    return _sc_gather_kernel()(a3, at, idxs)


_BM1 = 200


_BF = jnp.bfloat16


def _bdot(a, b):
    return jnp.dot(a.astype(_BF), b.astype(_BF),
                   preferred_element_type=jnp.float32)


def _comb(c0, v0, c1, v1):
    f32 = jnp.float32
    return (c0.astype(_BF).astype(f32) * v0.astype(_BF).astype(f32)
            + c1.astype(_BF).astype(f32) * v1.astype(_BF).astype(f32))


def _pass1_body(a0_ref, a1_ref, v1_ref, w1c_ref, b1_ref, gc1_ref, m_ref):
    i = pl.program_id(0)

    @pl.when(i == 0)
    def _init():
        for s in range(_SUPPORT):
            m_ref[s] = _comb(w1c_ref[s, 0], v1_ref[0],
                             w1c_ref[s, 1], v1_ref[1]).astype(_BF)

    acc = _bdot(a0_ref[0], m_ref[0])
    acc += _bdot(a1_ref[0], m_ref[1])
    gc1_ref[...] = jnp.maximum(acc + b1_ref[...], 0.0).astype(_BF)


def _pass1(A, V1, W1_comp, b1):
    grid = (_N // _BM1,)
    return pl.pallas_call(
        _pass1_body,
        grid=grid,
        in_specs=[
            pl.BlockSpec((1, _BM1, _N), lambda i: (0, i, 0)),
            pl.BlockSpec((1, _BM1, _N), lambda i: (1, i, 0)),
            pl.BlockSpec((_NUM_BASES, _N, _H), lambda i: (0, 0, 0)),
            pl.BlockSpec(memory_space=pltpu.SMEM),
            pl.BlockSpec((1, _H), lambda i: (0, 0)),
        ],
        out_specs=pl.BlockSpec((_BM1, _H), lambda i: (i, 0)),
        out_shape=jax.ShapeDtypeStruct((_N, _H), _BF),
        scratch_shapes=[
            pltpu.VMEM((_NUM_BASES, _N, _H), _BF),
        ],
        compiler_params=pltpu.CompilerParams(
            dimension_semantics=("arbitrary",)),
    )(A, A, V1, W1_comp, b1)


_BM2 = 128


def _pass2_body(ag0_ref, ag1_ref, gc1_ref, v2_ref, w2c_ref, b2_ref,
                wfc1_ref, bfc1_ref, wfc2_ref, bfc2_ref, out_ref):
    main = _NCH * _CW
    row = lax.broadcasted_iota(jnp.int32, (_TW, _H), 0)
    gt = jnp.where(row >= (_TW - (_N - main)), gc1_ref[_N - _TW:, :],
                   jnp.zeros((), _BF))
    e = jnp.zeros((_BM2, _H), jnp.float32)
    for s, ag_ref in ((0, ag0_ref), (1, ag1_ref)):
        ag = ag_ref[0]
        h = _bdot(ag[:, :main], gc1_ref[:main, :])
        h += _bdot(ag[:, main:], gt)
        w2e = _comb(w2c_ref[s, 0], v2_ref[0], w2c_ref[s, 1], v2_ref[1])
        e += _bdot(h, w2e)
    e = jnp.maximum(e + b2_ref[...], 0.0)
    f = jnp.maximum(_bdot(e, wfc1_ref[...]) + bfc1_ref[...], 0.0)
    logits = _bdot(f, wfc2_ref[...]) + bfc2_ref[...]
    mx = jnp.max(logits, axis=-1, keepdims=True)
    lse = mx + jnp.log(jnp.sum(jnp.exp(logits - mx), axis=-1, keepdims=True))
    out_ref[...] = logits - lse


def _pass2(Ag, gc1, V2, W2_comp, b2, Wfc1, bfc1, Wfc2, bfc2):
    grid = (_NUM_IDX // _BM2,)
    nclass = Wfc2.shape[1]
    return pl.pallas_call(
        _pass2_body,
        grid=grid,
        in_specs=[
            pl.BlockSpec((1, _BM2, _AGW), lambda i: (0, i, 0)),
            pl.BlockSpec((1, _BM2, _AGW), lambda i: (1, i, 0)),
            pl.BlockSpec((_N, _H), lambda i: (0, 0)),
            pl.BlockSpec((_NUM_BASES, _H, _H), lambda i: (0, 0, 0)),
            pl.BlockSpec(memory_space=pltpu.SMEM),
            pl.BlockSpec((1, _H), lambda i: (0, 0)),
            pl.BlockSpec((_H, _H), lambda i: (0, 0)),
            pl.BlockSpec((1, _H), lambda i: (0, 0)),
            pl.BlockSpec((_H, nclass), lambda i: (0, 0)),
            pl.BlockSpec((1, nclass), lambda i: (0, 0)),
        ],
        out_specs=pl.BlockSpec((_BM2, nclass), lambda i: (i, 0)),
        out_shape=jax.ShapeDtypeStruct((_NUM_IDX, nclass), jnp.float32),
        compiler_params=pltpu.CompilerParams(
            dimension_semantics=("arbitrary",)),
    )(Ag, Ag, gc1, V2, W2_comp, b2, Wfc1, bfc1, Wfc2, bfc2)


def kernel(vertex, A, idxs, W1, W1_comp, b1, W2, W2_comp, b2,
           Wfc1, bfc1, Wfc2, bfc2):
    del vertex
    idxs = idxs.astype(jnp.int32)
    nclass = Wfc2.shape[1]

    At = lax.slice(A, (0, 0, _N - _TW), (_SUPPORT, _N, _N))
    Ag = _sc_gather(A, At, idxs)
    V1 = W1.reshape(_NUM_BASES, _N, _H)
    V2 = W2.reshape(_NUM_BASES, _H, _H)
    gc1 = _pass1(A, V1, W1_comp, b1.reshape(1, _H))
    return _pass2(Ag, gc1, V2, W2_comp, b2.reshape(1, _H), Wfc1,
                  bfc1.reshape(1, _H), Wfc2, bfc2.reshape(1, nclass))
